# Initial kernel scaffold; baseline (speedup 1.0000x reference)
import jax
import jax.numpy as jnp
from jax.experimental import pallas as pl


def kernel(x, pos, batch, params):
    raise NotImplementedError("write your pallas kernel here")


# speedup vs baseline: 3.5255x; 3.5255x over previous
import functools

import jax
import jax.numpy as jnp
import numpy as np
from jax.experimental import pallas as pl


def _fps_body(px, py, pz, n_samples, iota_p, iota_s):
    Bb, Pp = px.shape
    Ss = n_samples

    def step(i, state):
        dists, far, qx, qy, qz = state
        onehot = (iota_p == far).astype(px.dtype)
        cx = jnp.sum(px * onehot, axis=1, keepdims=True)
        cy = jnp.sum(py * onehot, axis=1, keepdims=True)
        cz = jnp.sum(pz * onehot, axis=1, keepdims=True)
        qx = jnp.where(iota_s == i, cx, qx)
        qy = jnp.where(iota_s == i, cy, qy)
        qz = jnp.where(iota_s == i, cz, qz)
        dx = px - cx
        dy = py - cy
        dz = pz - cz
        d = dx * dx + dy * dy + dz * dz
        dists = jnp.minimum(dists, d)
        m = jnp.max(dists, axis=1, keepdims=True)
        far = jnp.min(jnp.where(dists == m, iota_p, Pp), axis=1, keepdims=True)
        return dists, far, qx, qy, qz

    init = (
        jnp.full((Bb, Pp), jnp.inf, dtype=px.dtype),
        jnp.zeros((Bb, 1), jnp.int32),
        jnp.zeros((Bb, Ss), px.dtype),
        jnp.zeros((Bb, Ss), px.dtype),
        jnp.zeros((Bb, Ss), px.dtype),
    )
    _, _, qx, qy, qz = jax.lax.fori_loop(0, Ss, step, init)
    return qx, qy, qz


def _fps_kernel(px_ref, py_ref, pz_ref,
                q1x_ref, q1y_ref, q1z_ref, q2x_ref, q2y_ref, q2z_ref,
                *, S1, S2):
    px, py, pz = px_ref[...], py_ref[...], pz_ref[...]
    Bb, Pp = px.shape
    iota_p1 = jax.lax.broadcasted_iota(jnp.int32, (Bb, Pp), 1)
    iota_s1 = jax.lax.broadcasted_iota(jnp.int32, (Bb, S1), 1)
    q1x, q1y, q1z = _fps_body(px, py, pz, S1, iota_p1, iota_s1)
    q1x_ref[...], q1y_ref[...], q1z_ref[...] = q1x, q1y, q1z
    iota_p2 = jax.lax.broadcasted_iota(jnp.int32, (Bb, S1), 1)
    iota_s2 = jax.lax.broadcasted_iota(jnp.int32, (Bb, S2), 1)
    q2x, q2y, q2z = _fps_body(q1x, q1y, q1z, S2, iota_p2, iota_s2)
    q2x_ref[...], q2y_ref[...], q2z_ref[...] = q2x, q2y, q2z


def _run_fps(pxyz, S1, S2, interpret=False):
    Bb, Pp, _ = pxyz.shape
    px, py, pz = pxyz[..., 0], pxyz[..., 1], pxyz[..., 2]
    outs = pl.pallas_call(
        functools.partial(_fps_kernel, S1=S1, S2=S2),
        out_shape=(
            jax.ShapeDtypeStruct((Bb, S1), jnp.float32),
            jax.ShapeDtypeStruct((Bb, S1), jnp.float32),
            jax.ShapeDtypeStruct((Bb, S1), jnp.float32),
            jax.ShapeDtypeStruct((Bb, S2), jnp.float32),
            jax.ShapeDtypeStruct((Bb, S2), jnp.float32),
            jax.ShapeDtypeStruct((Bb, S2), jnp.float32),
        ),
        interpret=interpret,
    )(px, py, pz)
    return outs


def _nbr_kernel(qx_ref, qy_ref, qz_ref, px_ref, py_ref, pz_ref,
                pen_ref, *, K, rsq):
    qx, qy, qz = qx_ref[0], qy_ref[0], qz_ref[0]
    px, py, pz = px_ref[0], py_ref[0], pz_ref[0]
    dx = qx - px
    dy = qy - py
    dz = qz - pz
    d2 = dx * dx + dy * dy + dz * dz
    bits = jax.lax.bitcast_convert_type(d2, jnp.int32)
    Ss = d2.shape[0]
    acc = jnp.zeros((Ss, 1), jnp.int32)
    for b in range(30, -1, -1):
        test = acc | (1 << b)
        cnt = jnp.sum((bits < test).astype(jnp.float32), axis=1, keepdims=True)
        acc = jnp.where(cnt >= K, acc, test)
    th = jnp.minimum(jax.lax.bitcast_convert_type(acc, jnp.float32),
                     jnp.float32(rsq))
    pen_ref[0] = jnp.where(d2 <= th, 0.0, -1e30)


def _run_nbr(q, p, K, rsq, interpret=False):
    Bb, Ss, _ = q.shape
    Pp = p.shape[1]
    qcols = [q[..., i][..., None] for i in range(3)]
    prows = [p[..., i][:, None, :] for i in range(3)]
    pen = pl.pallas_call(
        functools.partial(_nbr_kernel, K=K, rsq=rsq),
        grid=(Bb,),
        in_specs=[pl.BlockSpec((1, Ss, 1), lambda b: (b, 0, 0))] * 3
                 + [pl.BlockSpec((1, 1, Pp), lambda b: (b, 0, 0))] * 3,
        out_specs=pl.BlockSpec((1, Ss, Pp), lambda b: (b, 0, 0)),
        out_shape=jax.ShapeDtypeStruct((Bb, Ss, Pp), jnp.float32),
        interpret=interpret,
    )(*qcols, *prows)
    return pen


def _sa_dense_kernel(feat_ref, qf_ref, pen_ref,
                     w1_ref, w1p_ref, b1_ref, w2_ref, b2_ref, w3_ref, b3_ref,
                     out_ref):
    feat = feat_ref[0]
    u = jnp.dot(feat, w1_ref[...], preferred_element_type=jnp.float32)
    cc = b1_ref[...] - jnp.dot(qf_ref[0], w1p_ref[...],
                               preferred_element_type=jnp.float32)
    sc = cc.shape[0]
    Pp = u.shape[0]
    cout = out_ref.shape[-1]
    pair = cc[:, None, :] + u[None, :, :]
    z1 = jax.nn.relu(pair).reshape(sc * Pp, -1)
    z2 = jax.nn.relu(jnp.dot(z1, w2_ref[...],
                             preferred_element_type=jnp.float32)
                     + b2_ref[...])
    h = jnp.dot(z2, w3_ref[...],
                preferred_element_type=jnp.float32) + b3_ref[...]
    h = (h + pen_ref[0]).reshape(sc, Pp, cout)
    out_ref[0] = jnp.max(h, axis=1)


def _run_sa_dense(feat, qf, pen, layers, sc, din, interpret=False):
    Bb, Pp, Ff = feat.shape
    Ss = qf.shape[1]
    (w1, b1), (w2, b2), (w3, b3) = layers
    w1p = w1[din - 3:din + 1]
    cout = w3.shape[1]
    nt = Ss // sc
    out = pl.pallas_call(
        _sa_dense_kernel,
        grid=(Bb, nt),
        in_specs=[
            pl.BlockSpec((1, Pp, Ff), lambda b, t: (b, 0, 0)),
            pl.BlockSpec((1, sc, 4), lambda b, t: (b, t, 0)),
            pl.BlockSpec((1, sc * Pp, 1), lambda b, t: (b, t, 0)),
            pl.BlockSpec(w1.shape, lambda b, t: (0, 0)),
            pl.BlockSpec(w1p.shape, lambda b, t: (0, 0)),
            pl.BlockSpec(b1.shape, lambda b, t: (0, 0)),
            pl.BlockSpec(w2.shape, lambda b, t: (0, 0)),
            pl.BlockSpec(b2.shape, lambda b, t: (0, 0)),
            pl.BlockSpec(w3.shape, lambda b, t: (0, 0)),
            pl.BlockSpec(b3.shape, lambda b, t: (0, 0)),
        ],
        out_specs=pl.BlockSpec((1, sc, cout), lambda b, t: (b, t, 0)),
        out_shape=jax.ShapeDtypeStruct((Bb, Ss, cout), jnp.float32),
        interpret=interpret,
    )(feat, qf, pen, w1, w1p, b1, w2, b2, w3, b3)
    return out


def _tail_kernel(f3_ref, wa_ref, ba_ref, wb_ref, bb_ref, wc_ref, bc_ref,
                 wh1_ref, bh1_ref, wh2_ref, bh2_ref, wh3_ref, bh3_ref,
                 out_ref, *, B):
    f3 = f3_ref[...]
    h = jax.nn.relu(jnp.dot(f3, wa_ref[...],
                            preferred_element_type=jnp.float32) + ba_ref[...])
    h = jax.nn.relu(jnp.dot(h, wb_ref[...],
                            preferred_element_type=jnp.float32) + bb_ref[...])
    h = jnp.dot(h, wc_ref[...], preferred_element_type=jnp.float32) + bc_ref[...]
    n = f3.shape[0]
    g = jnp.max(h.reshape(B, n // B, -1), axis=1)
    g = jax.nn.relu(jnp.dot(g, wh1_ref[...],
                            preferred_element_type=jnp.float32) + bh1_ref[...])
    g = jax.nn.relu(jnp.dot(g, wh2_ref[...],
                            preferred_element_type=jnp.float32) + bh2_ref[...])
    out_ref[...] = jnp.dot(g, wh3_ref[...],
                           preferred_element_type=jnp.float32) + bh3_ref[...]


def _run_tail(f3, sa3, head, B, interpret=False):
    (wa, ba), (wb, bb), (wc, bc) = sa3
    (wh1, bh1), (wh2, bh2), (wh3, bh3) = head
    out = pl.pallas_call(
        functools.partial(_tail_kernel, B=B),
        out_shape=jax.ShapeDtypeStruct((B, head[2][0].shape[1]), jnp.float32),
        interpret=interpret,
    )(f3, wa, ba, wb, bb, wc, bc, wh1, bh1, wh2, bh2, wh3, bh3)
    return out


def _row(b):
    return b.reshape(1, -1)


def _forward_impl(x, pos, params, B, P, K=64, interpret=False):
    S1, S2 = P // 2, P // 8
    r1sq, r2sq = float(np.float32(0.2 * 0.2)), float(np.float32(0.4 * 0.4))
    xb = x.reshape(B, P, -1)
    pb = pos.reshape(B, P, 3)

    q1x, q1y, q1z, q2x, q2y, q2z = _run_fps(pb, S1, S2, interpret=interpret)
    q1 = jnp.stack([q1x, q1y, q1z], axis=-1)
    q2 = jnp.stack([q2x, q2y, q2z], axis=-1)

    pen1 = _run_nbr(q1, pb, K, r1sq, interpret=interpret)
    pen2 = _run_nbr(q2, q1, K, r2sq, interpret=interpret)
    pen1 = pen1.reshape(B, S1 * P, 1)
    pen2 = pen2.reshape(B, S2 * S1, 1)

    zpad = jnp.zeros((B, P, 2), jnp.float32)
    feat1 = jnp.concatenate([xb, pb, zpad], axis=-1)
    (w1, b1), (w2, b2), (w3, b3) = params['sa1']
    w1pad = jnp.concatenate([w1, jnp.zeros((2, w1.shape[1]), jnp.float32)], 0)
    q1f = jnp.concatenate([q1, jnp.zeros((B, S1, 1), jnp.float32)], axis=-1)
    lay1 = ((w1pad, _row(b1)), (w2, _row(b2)), (w3, _row(b3)))
    sc1 = min(S1, max(1, 8192 // P))
    x1 = _run_sa_dense(feat1, q1f, pen1, lay1, sc1, w1.shape[0],
                       interpret=interpret)

    (v1, c1), (v2, c2), (v3, c3) = params['sa2']
    f2 = v1.shape[0] + 5
    zpad2 = jnp.zeros((B, S1, f2 - x1.shape[-1] - 3), jnp.float32)
    feat2 = jnp.concatenate([x1, q1, zpad2], axis=-1)
    v1pad = jnp.concatenate([v1, jnp.zeros((5, v1.shape[1]), jnp.float32)], 0)
    q2f = jnp.concatenate([q2, jnp.zeros((B, S2, 1), jnp.float32)], axis=-1)
    lay2 = ((v1pad, _row(c1)), (v2, _row(c2)), (v3, _row(c3)))
    sc2 = min(S2, max(1, 8192 // S1))
    x2 = _run_sa_dense(feat2, q2f, pen2, lay2, sc2, v1.shape[0],
                       interpret=interpret)

    f3 = jnp.concatenate([x2, q2], axis=-1).reshape(B * S2, -1)
    return _run_tail(f3, [(w, _row(b)) for (w, b) in params['sa3']],
                     [(w, _row(b)) for (w, b) in params['head']],
                     B, interpret=interpret)


def kernel(x, pos, batch, params):
    B = 32
    P = x.shape[0] // B
    return _forward_impl(x, pos, params, B, P)


# speedup vs baseline: 3.9717x; 1.1266x over previous
import functools

import jax
import jax.numpy as jnp
import numpy as np
from jax.experimental import pallas as pl
from jax.experimental.pallas import tpu as pltpu
from jax.experimental.pallas import tpu_sc as plsc


def _fps_body(px, py, pz, n_samples, iota_p, iota_s):
    Bb, Pp = px.shape
    Ss = n_samples

    def step(i, state):
        dists, far, qx, qy, qz = state
        onehot = (iota_p == far).astype(px.dtype)
        cx = jnp.sum(px * onehot, axis=1, keepdims=True)
        cy = jnp.sum(py * onehot, axis=1, keepdims=True)
        cz = jnp.sum(pz * onehot, axis=1, keepdims=True)
        qx = jnp.where(iota_s == i, cx, qx)
        qy = jnp.where(iota_s == i, cy, qy)
        qz = jnp.where(iota_s == i, cz, qz)
        dx = px - cx
        dy = py - cy
        dz = pz - cz
        d = dx * dx + dy * dy + dz * dz
        dists = jnp.minimum(dists, d)
        m = jnp.max(dists, axis=1, keepdims=True)
        far = jnp.min(jnp.where(dists == m, iota_p, Pp), axis=1, keepdims=True)
        return dists, far, qx, qy, qz

    init = (
        jnp.full((Bb, Pp), jnp.inf, dtype=px.dtype),
        jnp.zeros((Bb, 1), jnp.int32),
        jnp.zeros((Bb, Ss), px.dtype),
        jnp.zeros((Bb, Ss), px.dtype),
        jnp.zeros((Bb, Ss), px.dtype),
    )
    _, _, qx, qy, qz = jax.lax.fori_loop(0, Ss, step, init)
    return qx, qy, qz


def _fps_kernel(px_ref, py_ref, pz_ref,
                q1x_ref, q1y_ref, q1z_ref, q2x_ref, q2y_ref, q2z_ref,
                *, S1, S2):
    px, py, pz = px_ref[...], py_ref[...], pz_ref[...]
    Bb, Pp = px.shape
    iota_p1 = jax.lax.broadcasted_iota(jnp.int32, (Bb, Pp), 1)
    iota_s1 = jax.lax.broadcasted_iota(jnp.int32, (Bb, S1), 1)
    q1x, q1y, q1z = _fps_body(px, py, pz, S1, iota_p1, iota_s1)
    q1x_ref[...], q1y_ref[...], q1z_ref[...] = q1x, q1y, q1z
    iota_p2 = jax.lax.broadcasted_iota(jnp.int32, (Bb, S1), 1)
    iota_s2 = jax.lax.broadcasted_iota(jnp.int32, (Bb, S2), 1)
    q2x, q2y, q2z = _fps_body(q1x, q1y, q1z, S2, iota_p2, iota_s2)
    q2x_ref[...], q2y_ref[...], q2z_ref[...] = q2x, q2y, q2z


def _run_fps(pxyz, S1, S2, interpret=False):
    Bb, Pp, _ = pxyz.shape
    px, py, pz = pxyz[..., 0], pxyz[..., 1], pxyz[..., 2]
    outs = pl.pallas_call(
        functools.partial(_fps_kernel, S1=S1, S2=S2),
        out_shape=(
            jax.ShapeDtypeStruct((Bb, S1), jnp.float32),
            jax.ShapeDtypeStruct((Bb, S1), jnp.float32),
            jax.ShapeDtypeStruct((Bb, S1), jnp.float32),
            jax.ShapeDtypeStruct((Bb, S2), jnp.float32),
            jax.ShapeDtypeStruct((Bb, S2), jnp.float32),
            jax.ShapeDtypeStruct((Bb, S2), jnp.float32),
        ),
        interpret=interpret,
    )(px, py, pz)
    return outs


def _nbr_kernel(qx_ref, qy_ref, qz_ref, px_ref, py_ref, pz_ref,
                pen_ref, *, K, rsq):
    qx, qy, qz = qx_ref[0], qy_ref[0], qz_ref[0]
    px, py, pz = px_ref[0], py_ref[0], pz_ref[0]
    dx = qx - px
    dy = qy - py
    dz = qz - pz
    d2 = dx * dx + dy * dy + dz * dz
    bits = jax.lax.bitcast_convert_type(d2, jnp.int32)
    Ss = d2.shape[0]
    acc = jnp.zeros((Ss, 1), jnp.int32)
    for b in range(30, -1, -1):
        test = acc | (1 << b)
        cnt = jnp.sum((bits < test).astype(jnp.float32), axis=1, keepdims=True)
        acc = jnp.where(cnt >= K, acc, test)
    th = jnp.minimum(jax.lax.bitcast_convert_type(acc, jnp.float32),
                     jnp.float32(rsq))
    pen_ref[0] = jnp.where(d2 <= th, 0.0, -1e30)


def _run_nbr(q, p, K, rsq, interpret=False):
    Bb, Ss, _ = q.shape
    Pp = p.shape[1]
    qcols = [q[..., i][..., None] for i in range(3)]
    prows = [p[..., i][:, None, :] for i in range(3)]
    pen = pl.pallas_call(
        functools.partial(_nbr_kernel, K=K, rsq=rsq),
        grid=(Bb,),
        in_specs=[pl.BlockSpec((1, Ss, 1), lambda b: (b, 0, 0))] * 3
                 + [pl.BlockSpec((1, 1, Pp), lambda b: (b, 0, 0))] * 3,
        out_specs=pl.BlockSpec((1, Ss, Pp), lambda b: (b, 0, 0)),
        out_shape=jax.ShapeDtypeStruct((Bb, Ss, Pp), jnp.float32),
        interpret=interpret,
    )(*qcols, *prows)
    return pen


def _sa_dense_kernel(feat_ref, qf_ref, pen_ref,
                     w1_ref, w1p_ref, b1_ref, w2_ref, b2_ref, w3_ref, b3_ref,
                     out_ref):
    feat = feat_ref[0]
    u = jnp.dot(feat, w1_ref[...], preferred_element_type=jnp.float32)
    cc = b1_ref[...] - jnp.dot(qf_ref[0], w1p_ref[...],
                               preferred_element_type=jnp.float32)
    sc = cc.shape[0]
    Pp = u.shape[0]
    cout = out_ref.shape[-1]
    pair = cc[:, None, :] + u[None, :, :]
    z1 = jax.nn.relu(pair).reshape(sc * Pp, -1)
    z2 = jax.nn.relu(jnp.dot(z1, w2_ref[...],
                             preferred_element_type=jnp.float32)
                     + b2_ref[...])
    h = jnp.dot(z2, w3_ref[...],
                preferred_element_type=jnp.float32) + b3_ref[...]
    h = (h + pen_ref[0]).reshape(sc, Pp, cout)
    out_ref[0] = jnp.max(h, axis=1)


def _run_sa_dense(feat, qf, pen, layers, sc, din, interpret=False):
    Bb, Pp, Ff = feat.shape
    Ss = qf.shape[1]
    (w1, b1), (w2, b2), (w3, b3) = layers
    w1p = w1[din - 3:din + 1]
    cout = w3.shape[1]
    nt = Ss // sc
    out = pl.pallas_call(
        _sa_dense_kernel,
        grid=(Bb, nt),
        in_specs=[
            pl.BlockSpec((1, Pp, Ff), lambda b, t: (b, 0, 0)),
            pl.BlockSpec((1, sc, 4), lambda b, t: (b, t, 0)),
            pl.BlockSpec((1, sc * Pp, 1), lambda b, t: (b, t, 0)),
            pl.BlockSpec(w1.shape, lambda b, t: (0, 0)),
            pl.BlockSpec(w1p.shape, lambda b, t: (0, 0)),
            pl.BlockSpec(b1.shape, lambda b, t: (0, 0)),
            pl.BlockSpec(w2.shape, lambda b, t: (0, 0)),
            pl.BlockSpec(b2.shape, lambda b, t: (0, 0)),
            pl.BlockSpec(w3.shape, lambda b, t: (0, 0)),
            pl.BlockSpec(b3.shape, lambda b, t: (0, 0)),
        ],
        out_specs=pl.BlockSpec((1, sc, cout), lambda b, t: (b, t, 0)),
        out_shape=jax.ShapeDtypeStruct((Bb, Ss, cout), jnp.float32),
        interpret=interpret,
    )(feat, qf, pen, w1, w1p, b1, w2, b2, w3, b3)
    return out


_SC_CORES, _SC_SUBCORES = 2, 16


def _sc_compact_gather(pen2d, feats, S, P, K):
    R = pen2d.shape[0]
    NW = _SC_CORES * _SC_SUBCORES
    rpw = R // NW
    rpc = R // _SC_CORES
    mesh = plsc.VectorSubcoreMesh(core_axis_name="c", subcore_axis_name="s")
    Fs = [f.shape[1] for f in feats]
    for F in Fs:
        assert F % 128 == 0, F

    scratch = [pltpu.VMEM((K, F), jnp.float32) for F in Fs]
    scratch += [
        pltpu.VMEM((P,), jnp.float32),
        pltpu.VMEM((P,), jnp.int32),
        pltpu.VMEM((K,), jnp.int32),
        pltpu.VMEM((rpw,), jnp.int32),
        pltpu.SemaphoreType.DMA,
    ]

    @functools.partial(
        pl.kernel,
        out_type=tuple([jax.ShapeDtypeStruct((R * K, F), jnp.float32)
                        for F in Fs]
                       + [jax.ShapeDtypeStruct((R,), jnp.int32)]),
        mesh=mesh,
        scratch_types=scratch,
        compiler_params=pltpu.CompilerParams(needs_layout_passes=False),
    )
    def k(pen_hbm, *rest):
        nf = len(Fs)
        feat_hbms = rest[:nf]
        out_hbms = rest[nf:2 * nf]
        cnt_hbm = rest[2 * nf]
        rows_vs = rest[2 * nf + 1:3 * nf + 1]
        pen_v, idx_v, idxk_v, cnt_v, sem = rest[3 * nf + 1:]
        core = jax.lax.axis_index("c")
        sub = jax.lax.axis_index("s")
        r0 = core * rpc + sub * rpw
        lane = jax.lax.iota(jnp.int32, 16)
        zeros = jnp.zeros((16,), jnp.int32)
        for t in range(K // 16):
            idx_v[pl.ds(t * 16, 16)] = zeros

        def row_body(i, carry):
            r = r0 + i
            b = r // S
            jbase = b * P
            pltpu.sync_copy(pen_hbm.at[r], pen_v)
            cnt_splat = zeros
            for c2 in range(P // 16):
                v = pen_v[pl.ds(c2 * 16, 16)]
                m = v > jnp.float32(-1.0)
                pos = cnt_splat + plsc.cumsum(m.astype(jnp.int32)) - 1
                vals = lane + (jbase + c2 * 16)
                plsc.store_scatter(idx_v, [pos], vals, mask=m)
                cnt_splat = cnt_splat + plsc.all_reduce_population_count(m)
            plsc.store_scatter(cnt_v, [lane * 0 + i], cnt_splat,
                               mask=lane < 1)
            plsc.subcore_barrier()
            for t in range(K // 16):
                idxk_v[pl.ds(t * 16, 16)] = idx_v[pl.ds(t * 16, 16)]
            for fi in range(nf):
                pltpu.async_copy(feat_hbms[fi].at[idxk_v],
                                 rows_vs[fi], sem).wait()
                pltpu.sync_copy(rows_vs[fi],
                                out_hbms[fi].at[pl.ds(r * K, K)])
            return carry

        jax.lax.fori_loop(0, rpw, row_body, 0)
        pltpu.sync_copy(cnt_v, cnt_hbm.at[pl.ds(r0, rpw)])

    outs = k(pen2d, *feats)
    return list(outs[:-1]), outs[-1]


def _sa_sparse_kernel(*refs, K, nf):
    g_refs = refs[:nf]
    qf_ref, cnt_ref = refs[nf], refs[nf + 1]
    w_refs = refs[nf + 2:2 * nf + 2]
    (w1p_ref, b1_ref, w2_ref, b2_ref, w3_ref, b3_ref, out_ref) = refs[2 * nf + 2:]
    u = jnp.dot(g_refs[0][0], w_refs[0][...],
                preferred_element_type=jnp.float32)
    for i in range(1, nf):
        u = u + jnp.dot(g_refs[i][0], w_refs[i][...],
                        preferred_element_type=jnp.float32)
    c = b1_ref[...] - jnp.dot(qf_ref[0], w1p_ref[...],
                              preferred_element_type=jnp.float32)
    scq = c.shape[0]
    h1 = c.shape[1]
    cout = out_ref.shape[-1]
    cexp = jnp.broadcast_to(c[:, None, :], (scq, K, h1)).reshape(scq * K, h1)
    z1 = jax.nn.relu(u + cexp)
    z2 = jax.nn.relu(jnp.dot(z1, w2_ref[...],
                             preferred_element_type=jnp.float32) + b2_ref[...])
    h = jnp.dot(z2, w3_ref[...],
                preferred_element_type=jnp.float32) + b3_ref[...]
    rio = jax.lax.broadcasted_iota(jnp.int32, (scq * K, 1), 0)
    k_of = jax.lax.rem(rio, K)
    cnt = cnt_ref[0]
    cnt_exp = jnp.broadcast_to(cnt[:, None, :], (scq, K, 1)).reshape(scq * K, 1)
    pen = jnp.where(k_of < cnt_exp, 0.0, -1e30)
    h = (h + pen).reshape(scq, K, cout)
    out_ref[0] = jnp.max(h, axis=1)


def _run_sa_sparse(gs, qf, cnt, ws, w1p, b1, w2, b2, w3, b3, sc, K,
                   interpret=False):
    nf = len(gs)
    Bb = qf.shape[0]
    Ss = qf.shape[1]
    cout = w3.shape[1]
    nt = Ss // sc
    gspecs = [pl.BlockSpec((1, sc * K, g.shape[2]), lambda b, t: (b, t, 0))
              for g in gs]
    wspecs = [pl.BlockSpec(w.shape, lambda b, t: (0, 0)) for w in ws]
    out = pl.pallas_call(
        functools.partial(_sa_sparse_kernel, K=K, nf=nf),
        grid=(Bb, nt),
        in_specs=gspecs + [
            pl.BlockSpec((1, sc, 4), lambda b, t: (b, t, 0)),
            pl.BlockSpec((1, sc, 1), lambda b, t: (b, t, 0)),
        ] + wspecs + [
            pl.BlockSpec(w1p.shape, lambda b, t: (0, 0)),
            pl.BlockSpec(b1.shape, lambda b, t: (0, 0)),
            pl.BlockSpec(w2.shape, lambda b, t: (0, 0)),
            pl.BlockSpec(b2.shape, lambda b, t: (0, 0)),
            pl.BlockSpec(w3.shape, lambda b, t: (0, 0)),
            pl.BlockSpec(b3.shape, lambda b, t: (0, 0)),
        ],
        out_specs=pl.BlockSpec((1, sc, cout), lambda b, t: (b, t, 0)),
        out_shape=jax.ShapeDtypeStruct((Bb, Ss, cout), jnp.float32),
        interpret=interpret,
    )(*gs, qf, cnt, *ws, w1p, b1, w2, b2, w3, b3)
    return out


def _tail_kernel(f3_ref, wa_ref, ba_ref, wb_ref, bb_ref, wc_ref, bc_ref,
                 wh1_ref, bh1_ref, wh2_ref, bh2_ref, wh3_ref, bh3_ref,
                 out_ref, *, B):
    f3 = f3_ref[...]
    h = jax.nn.relu(jnp.dot(f3, wa_ref[...],
                            preferred_element_type=jnp.float32) + ba_ref[...])
    h = jax.nn.relu(jnp.dot(h, wb_ref[...],
                            preferred_element_type=jnp.float32) + bb_ref[...])
    h = jnp.dot(h, wc_ref[...], preferred_element_type=jnp.float32) + bc_ref[...]
    n = f3.shape[0]
    g = jnp.max(h.reshape(B, n // B, -1), axis=1)
    g = jax.nn.relu(jnp.dot(g, wh1_ref[...],
                            preferred_element_type=jnp.float32) + bh1_ref[...])
    g = jax.nn.relu(jnp.dot(g, wh2_ref[...],
                            preferred_element_type=jnp.float32) + bh2_ref[...])
    out_ref[...] = jnp.dot(g, wh3_ref[...],
                           preferred_element_type=jnp.float32) + bh3_ref[...]


def _run_tail(f3, sa3, head, B, interpret=False):
    (wa, ba), (wb, bb), (wc, bc) = sa3
    (wh1, bh1), (wh2, bh2), (wh3, bh3) = head
    out = pl.pallas_call(
        functools.partial(_tail_kernel, B=B),
        out_shape=jax.ShapeDtypeStruct((B, head[2][0].shape[1]), jnp.float32),
        interpret=interpret,
    )(f3, wa, ba, wb, bb, wc, bc, wh1, bh1, wh2, bh2, wh3, bh3)
    return out


def _row(b):
    return b.reshape(1, -1)


def _forward_impl(x, pos, params, B, P, K=64, interpret=False):
    S1, S2 = P // 2, P // 8
    r1sq, r2sq = float(np.float32(0.2 * 0.2)), float(np.float32(0.4 * 0.4))
    xb = x.reshape(B, P, -1)
    pb = pos.reshape(B, P, 3)

    q1x, q1y, q1z, q2x, q2y, q2z = _run_fps(pb, S1, S2, interpret=interpret)
    q1 = jnp.stack([q1x, q1y, q1z], axis=-1)
    q2 = jnp.stack([q2x, q2y, q2z], axis=-1)

    pen1 = _run_nbr(q1, pb, K, r1sq, interpret=interpret)
    pen2 = _run_nbr(q2, q1, K, r2sq, interpret=interpret)

    f1 = 128
    zpad = jnp.zeros((B, P, f1 - 6), jnp.float32)
    feat1 = jnp.concatenate([xb, pb, zpad], axis=-1)
    (w1, b1), (w2, b2), (w3, b3) = params['sa1']
    w1pad = jnp.concatenate(
        [w1, jnp.zeros((f1 - 6, w1.shape[1]), jnp.float32)], 0)
    w1p = w1pad[3:7]
    q1f = jnp.concatenate([q1, jnp.zeros((B, S1, 1), jnp.float32)], axis=-1)
    (g1,), cnt1 = _sc_compact_gather(pen1.reshape(B * S1, P),
                                     [feat1.reshape(B * P, f1)], S1, P, K)
    x1 = _run_sa_sparse([g1.reshape(B, S1 * K, f1)], q1f,
                        cnt1.reshape(B, S1, 1), [w1pad], w1p,
                        _row(b1), w2, _row(b2), w3, _row(b3), min(S1, 64), K,
                        interpret=interpret)

    (v1, c1), (v2, c2), (v3, c3) = params['sa2']
    pos2 = jnp.concatenate([q1, jnp.zeros((B, S1, 125), jnp.float32)],
                           axis=-1)
    cdim = x1.shape[-1]
    v1x = v1[:cdim]
    v1p = jnp.concatenate([v1[cdim:], jnp.zeros((1, v1.shape[1]),
                                                jnp.float32)], 0)
    v1p128 = jnp.concatenate([v1[cdim:], jnp.zeros((125, v1.shape[1]),
                                                   jnp.float32)], 0)
    q2f = jnp.concatenate([q2, jnp.zeros((B, S2, 1), jnp.float32)], axis=-1)
    (g2a, g2b), cnt2 = _sc_compact_gather(
        pen2.reshape(B * S2, S1),
        [x1.reshape(B * S1, cdim), pos2.reshape(B * S1, 128)], S2, S1, K)
    x2 = _run_sa_sparse([g2a.reshape(B, S2 * K, cdim),
                         g2b.reshape(B, S2 * K, 128)], q2f,
                        cnt2.reshape(B, S2, 1), [v1x, v1p128], v1p,
                        _row(c1), v2, _row(c2), v3, _row(c3), min(S2, 32), K,
                        interpret=interpret)

    f3 = jnp.concatenate([x2, q2], axis=-1).reshape(B * S2, -1)
    return _run_tail(f3, [(w, _row(b)) for (w, b) in params['sa3']],
                     [(w, _row(b)) for (w, b) in params['head']],
                     B, interpret=interpret)


def kernel(x, pos, batch, params):
    B = 32
    P = x.shape[0] // B
    return _forward_impl(x, pos, params, B, P)


# speedup vs baseline: 4.0310x; 1.0149x over previous
import functools

import jax
import jax.numpy as jnp
import numpy as np
from jax.experimental import pallas as pl
from jax.experimental.pallas import tpu as pltpu
from jax.experimental.pallas import tpu_sc as plsc


def _fps_body(px, py, pz, n_samples, iota_p, iota_s):
    Bb, Pp = px.shape
    Ss = n_samples

    def step(i, state):
        dists, far, qx, qy, qz = state
        onehot = (iota_p == far).astype(px.dtype)
        cx = jnp.sum(px * onehot, axis=1, keepdims=True)
        cy = jnp.sum(py * onehot, axis=1, keepdims=True)
        cz = jnp.sum(pz * onehot, axis=1, keepdims=True)
        qx = jnp.where(iota_s == i, cx, qx)
        qy = jnp.where(iota_s == i, cy, qy)
        qz = jnp.where(iota_s == i, cz, qz)
        dx = px - cx
        dy = py - cy
        dz = pz - cz
        d = dx * dx + dy * dy + dz * dz
        dists = jnp.minimum(dists, d)
        m = jnp.max(dists, axis=1, keepdims=True)
        far = jnp.min(jnp.where(dists == m, iota_p, Pp), axis=1, keepdims=True)
        return dists, far, qx, qy, qz

    init = (
        jnp.full((Bb, Pp), jnp.inf, dtype=px.dtype),
        jnp.zeros((Bb, 1), jnp.int32),
        jnp.zeros((Bb, Ss), px.dtype),
        jnp.zeros((Bb, Ss), px.dtype),
        jnp.zeros((Bb, Ss), px.dtype),
    )
    _, _, qx, qy, qz = jax.lax.fori_loop(0, Ss, step, init)
    return qx, qy, qz


def _fps_kernel(px_ref, py_ref, pz_ref,
                q1x_ref, q1y_ref, q1z_ref, q2x_ref, q2y_ref, q2z_ref,
                *, S1, S2):
    px, py, pz = px_ref[...], py_ref[...], pz_ref[...]
    Bb, Pp = px.shape
    iota_p1 = jax.lax.broadcasted_iota(jnp.int32, (Bb, Pp), 1)
    iota_s1 = jax.lax.broadcasted_iota(jnp.int32, (Bb, S1), 1)
    q1x, q1y, q1z = _fps_body(px, py, pz, S1, iota_p1, iota_s1)
    q1x_ref[...], q1y_ref[...], q1z_ref[...] = q1x, q1y, q1z
    iota_p2 = jax.lax.broadcasted_iota(jnp.int32, (Bb, S1), 1)
    iota_s2 = jax.lax.broadcasted_iota(jnp.int32, (Bb, S2), 1)
    q2x, q2y, q2z = _fps_body(q1x, q1y, q1z, S2, iota_p2, iota_s2)
    q2x_ref[...], q2y_ref[...], q2z_ref[...] = q2x, q2y, q2z


def _run_fps(pxyz, S1, S2, interpret=False):
    Bb, Pp, _ = pxyz.shape
    px, py, pz = pxyz[..., 0], pxyz[..., 1], pxyz[..., 2]
    outs = pl.pallas_call(
        functools.partial(_fps_kernel, S1=S1, S2=S2),
        out_shape=(
            jax.ShapeDtypeStruct((Bb, S1), jnp.float32),
            jax.ShapeDtypeStruct((Bb, S1), jnp.float32),
            jax.ShapeDtypeStruct((Bb, S1), jnp.float32),
            jax.ShapeDtypeStruct((Bb, S2), jnp.float32),
            jax.ShapeDtypeStruct((Bb, S2), jnp.float32),
            jax.ShapeDtypeStruct((Bb, S2), jnp.float32),
        ),
        interpret=interpret,
    )(px, py, pz)
    return outs


def _nbr_kernel(qx_ref, qy_ref, qz_ref, px_ref, py_ref, pz_ref,
                pen_ref, *, K, rsq):
    qx, qy, qz = qx_ref[0], qy_ref[0], qz_ref[0]
    px, py, pz = px_ref[0], py_ref[0], pz_ref[0]
    dx = qx - px
    dy = qy - py
    dz = qz - pz
    d2 = dx * dx + dy * dy + dz * dz
    Ss = d2.shape[0]
    inr = d2 <= jnp.float32(rsq)
    pen_ref[0] = jnp.where(inr, 0.0, -1e30)
    cnt_r = jnp.sum(inr.astype(jnp.float32), axis=1, keepdims=True)

    @pl.when(jnp.max(cnt_r) > K)
    def _():
        bits = jax.lax.bitcast_convert_type(d2, jnp.int32)
        acc = jnp.zeros((Ss, 1), jnp.int32)
        for b in range(30, -1, -1):
            test = acc | (1 << b)
            cnt = jnp.sum((bits < test).astype(jnp.float32), axis=1,
                          keepdims=True)
            acc = jnp.where(cnt >= K, acc, test)
        th = jnp.minimum(jax.lax.bitcast_convert_type(acc, jnp.float32),
                         jnp.float32(rsq))
        pen_ref[0] = jnp.where(d2 <= th, 0.0, -1e30)


def _run_nbr(q, p, K, rsq, interpret=False):
    Bb, Ss, _ = q.shape
    Pp = p.shape[1]
    qcols = [q[..., i][..., None] for i in range(3)]
    prows = [p[..., i][:, None, :] for i in range(3)]
    pen = pl.pallas_call(
        functools.partial(_nbr_kernel, K=K, rsq=rsq),
        grid=(Bb,),
        in_specs=[pl.BlockSpec((1, Ss, 1), lambda b: (b, 0, 0))] * 3
                 + [pl.BlockSpec((1, 1, Pp), lambda b: (b, 0, 0))] * 3,
        out_specs=pl.BlockSpec((1, Ss, Pp), lambda b: (b, 0, 0)),
        out_shape=jax.ShapeDtypeStruct((Bb, Ss, Pp), jnp.float32),
        interpret=interpret,
    )(*qcols, *prows)
    return pen


def _sa_dense_kernel(feat_ref, qf_ref, pen_ref,
                     w1_ref, w1p_ref, b1_ref, w2_ref, b2_ref, w3_ref, b3_ref,
                     out_ref):
    feat = feat_ref[0]
    u = jnp.dot(feat, w1_ref[...], preferred_element_type=jnp.float32)
    cc = b1_ref[...] - jnp.dot(qf_ref[0], w1p_ref[...],
                               preferred_element_type=jnp.float32)
    sc = cc.shape[0]
    Pp = u.shape[0]
    cout = out_ref.shape[-1]
    pair = cc[:, None, :] + u[None, :, :]
    z1 = jax.nn.relu(pair).reshape(sc * Pp, -1)
    z2 = jax.nn.relu(jnp.dot(z1, w2_ref[...],
                             preferred_element_type=jnp.float32)
                     + b2_ref[...])
    h = jnp.dot(z2, w3_ref[...],
                preferred_element_type=jnp.float32) + b3_ref[...]
    h = (h + pen_ref[0]).reshape(sc, Pp, cout)
    out_ref[0] = jnp.max(h, axis=1)


def _run_sa_dense(feat, qf, pen, layers, sc, din, interpret=False):
    Bb, Pp, Ff = feat.shape
    Ss = qf.shape[1]
    (w1, b1), (w2, b2), (w3, b3) = layers
    w1p = w1[din - 3:din + 1]
    cout = w3.shape[1]
    nt = Ss // sc
    out = pl.pallas_call(
        _sa_dense_kernel,
        grid=(Bb, nt),
        in_specs=[
            pl.BlockSpec((1, Pp, Ff), lambda b, t: (b, 0, 0)),
            pl.BlockSpec((1, sc, 4), lambda b, t: (b, t, 0)),
            pl.BlockSpec((1, sc * Pp, 1), lambda b, t: (b, t, 0)),
            pl.BlockSpec(w1.shape, lambda b, t: (0, 0)),
            pl.BlockSpec(w1p.shape, lambda b, t: (0, 0)),
            pl.BlockSpec(b1.shape, lambda b, t: (0, 0)),
            pl.BlockSpec(w2.shape, lambda b, t: (0, 0)),
            pl.BlockSpec(b2.shape, lambda b, t: (0, 0)),
            pl.BlockSpec(w3.shape, lambda b, t: (0, 0)),
            pl.BlockSpec(b3.shape, lambda b, t: (0, 0)),
        ],
        out_specs=pl.BlockSpec((1, sc, cout), lambda b, t: (b, t, 0)),
        out_shape=jax.ShapeDtypeStruct((Bb, Ss, cout), jnp.float32),
        interpret=interpret,
    )(feat, qf, pen, w1, w1p, b1, w2, b2, w3, b3)
    return out


_SC_CORES, _SC_SUBCORES = 2, 16


def _sc_compact_gather(pen2d, feats, S, P, K):
    R = pen2d.shape[0]
    NW = _SC_CORES * _SC_SUBCORES
    rpw = R // NW
    rpc = R // _SC_CORES
    mesh = plsc.VectorSubcoreMesh(core_axis_name="c", subcore_axis_name="s")
    Fs = [f.shape[1] for f in feats]
    for F in Fs:
        assert F % 128 == 0, F

    scratch = [pltpu.VMEM((K, F), jnp.float32) for F in Fs]
    scratch += [
        pltpu.VMEM((P,), jnp.float32),
        pltpu.VMEM((P,), jnp.int32),
        pltpu.VMEM((K,), jnp.int32),
        pltpu.VMEM((rpw,), jnp.int32),
        pltpu.SemaphoreType.DMA,
    ]

    @functools.partial(
        pl.kernel,
        out_type=tuple([jax.ShapeDtypeStruct((R * K, F), jnp.float32)
                        for F in Fs]
                       + [jax.ShapeDtypeStruct((R,), jnp.int32)]),
        mesh=mesh,
        scratch_types=scratch,
        compiler_params=pltpu.CompilerParams(needs_layout_passes=False),
    )
    def k(pen_hbm, *rest):
        nf = len(Fs)
        feat_hbms = rest[:nf]
        out_hbms = rest[nf:2 * nf]
        cnt_hbm = rest[2 * nf]
        rows_vs = rest[2 * nf + 1:3 * nf + 1]
        pen_v, idx_v, idxk_v, cnt_v, sem = rest[3 * nf + 1:]
        core = jax.lax.axis_index("c")
        sub = jax.lax.axis_index("s")
        r0 = core * rpc + sub * rpw
        lane = jax.lax.iota(jnp.int32, 16)
        zeros = jnp.zeros((16,), jnp.int32)
        for t in range(K // 16):
            idx_v[pl.ds(t * 16, 16)] = zeros

        def row_body(i, carry):
            r = r0 + i
            b = r // S
            jbase = b * P
            pltpu.sync_copy(pen_hbm.at[r], pen_v)
            cnt_splat = zeros
            for c2 in range(P // 16):
                v = pen_v[pl.ds(c2 * 16, 16)]
                m = v > jnp.float32(-1.0)
                pos = cnt_splat + plsc.cumsum(m.astype(jnp.int32)) - 1
                vals = lane + (jbase + c2 * 16)
                plsc.store_scatter(idx_v, [pos], vals, mask=m)
                cnt_splat = cnt_splat + plsc.all_reduce_population_count(m)
            plsc.store_scatter(cnt_v, [lane * 0 + i], cnt_splat,
                               mask=lane < 1)
            plsc.subcore_barrier()
            for t in range(K // 16):
                idxk_v[pl.ds(t * 16, 16)] = idx_v[pl.ds(t * 16, 16)]
            for fi in range(nf):
                pltpu.async_copy(feat_hbms[fi].at[idxk_v],
                                 rows_vs[fi], sem).wait()
                pltpu.sync_copy(rows_vs[fi],
                                out_hbms[fi].at[pl.ds(r * K, K)])
            return carry

        jax.lax.fori_loop(0, rpw, row_body, 0)
        pltpu.sync_copy(cnt_v, cnt_hbm.at[pl.ds(r0, rpw)])

    outs = k(pen2d, *feats)
    return list(outs[:-1]), outs[-1]


def _sa_sparse_kernel(*refs, K, nf):
    g_refs = refs[:nf]
    qf_ref, cnt_ref = refs[nf], refs[nf + 1]
    w_refs = refs[nf + 2:2 * nf + 2]
    (w1p_ref, b1_ref, w2_ref, b2_ref, w3_ref, b3_ref, out_ref) = refs[2 * nf + 2:]
    u = jnp.dot(g_refs[0][0], w_refs[0][...],
                preferred_element_type=jnp.float32)
    for i in range(1, nf):
        u = u + jnp.dot(g_refs[i][0], w_refs[i][...],
                        preferred_element_type=jnp.float32)
    c = b1_ref[...] - jnp.dot(qf_ref[0], w1p_ref[...],
                              preferred_element_type=jnp.float32)
    scq = c.shape[0]
    h1 = c.shape[1]
    cout = out_ref.shape[-1]
    cexp = jnp.broadcast_to(c[:, None, :], (scq, K, h1)).reshape(scq * K, h1)
    z1 = jax.nn.relu(u + cexp)
    z2 = jax.nn.relu(jnp.dot(z1, w2_ref[...],
                             preferred_element_type=jnp.float32) + b2_ref[...])
    h = jnp.dot(z2, w3_ref[...],
                preferred_element_type=jnp.float32) + b3_ref[...]
    rio = jax.lax.broadcasted_iota(jnp.int32, (scq * K, 1), 0)
    k_of = jax.lax.rem(rio, K)
    cnt = cnt_ref[0]
    cnt_exp = jnp.broadcast_to(cnt[:, None, :], (scq, K, 1)).reshape(scq * K, 1)
    pen = jnp.where(k_of < cnt_exp, 0.0, -1e30)
    h = (h + pen).reshape(scq, K, cout)
    out_ref[0] = jnp.max(h, axis=1)


def _run_sa_sparse(gs, qf, cnt, ws, w1p, b1, w2, b2, w3, b3, sc, K,
                   interpret=False):
    nf = len(gs)
    Bb = qf.shape[0]
    Ss = qf.shape[1]
    cout = w3.shape[1]
    nt = Ss // sc
    gspecs = [pl.BlockSpec((1, sc * K, g.shape[2]), lambda b, t: (b, t, 0))
              for g in gs]
    wspecs = [pl.BlockSpec(w.shape, lambda b, t: (0, 0)) for w in ws]
    out = pl.pallas_call(
        functools.partial(_sa_sparse_kernel, K=K, nf=nf),
        grid=(Bb, nt),
        in_specs=gspecs + [
            pl.BlockSpec((1, sc, 4), lambda b, t: (b, t, 0)),
            pl.BlockSpec((1, sc, 1), lambda b, t: (b, t, 0)),
        ] + wspecs + [
            pl.BlockSpec(w1p.shape, lambda b, t: (0, 0)),
            pl.BlockSpec(b1.shape, lambda b, t: (0, 0)),
            pl.BlockSpec(w2.shape, lambda b, t: (0, 0)),
            pl.BlockSpec(b2.shape, lambda b, t: (0, 0)),
            pl.BlockSpec(w3.shape, lambda b, t: (0, 0)),
            pl.BlockSpec(b3.shape, lambda b, t: (0, 0)),
        ],
        out_specs=pl.BlockSpec((1, sc, cout), lambda b, t: (b, t, 0)),
        out_shape=jax.ShapeDtypeStruct((Bb, Ss, cout), jnp.float32),
        interpret=interpret,
    )(*gs, qf, cnt, *ws, w1p, b1, w2, b2, w3, b3)
    return out


def _tail_kernel(f3_ref, wa_ref, ba_ref, wb_ref, bb_ref, wc_ref, bc_ref,
                 wh1_ref, bh1_ref, wh2_ref, bh2_ref, wh3_ref, bh3_ref,
                 out_ref, *, B):
    f3 = f3_ref[...]
    h = jax.nn.relu(jnp.dot(f3, wa_ref[...],
                            preferred_element_type=jnp.float32) + ba_ref[...])
    h = jax.nn.relu(jnp.dot(h, wb_ref[...],
                            preferred_element_type=jnp.float32) + bb_ref[...])
    h = jnp.dot(h, wc_ref[...], preferred_element_type=jnp.float32) + bc_ref[...]
    n = f3.shape[0]
    g = jnp.max(h.reshape(B, n // B, -1), axis=1)
    g = jax.nn.relu(jnp.dot(g, wh1_ref[...],
                            preferred_element_type=jnp.float32) + bh1_ref[...])
    g = jax.nn.relu(jnp.dot(g, wh2_ref[...],
                            preferred_element_type=jnp.float32) + bh2_ref[...])
    out_ref[...] = jnp.dot(g, wh3_ref[...],
                           preferred_element_type=jnp.float32) + bh3_ref[...]


def _run_tail(f3, sa3, head, B, interpret=False):
    (wa, ba), (wb, bb), (wc, bc) = sa3
    (wh1, bh1), (wh2, bh2), (wh3, bh3) = head
    out = pl.pallas_call(
        functools.partial(_tail_kernel, B=B),
        out_shape=jax.ShapeDtypeStruct((B, head[2][0].shape[1]), jnp.float32),
        interpret=interpret,
    )(f3, wa, ba, wb, bb, wc, bc, wh1, bh1, wh2, bh2, wh3, bh3)
    return out


def _row(b):
    return b.reshape(1, -1)


def _forward_impl(x, pos, params, B, P, K=64, interpret=False):
    S1, S2 = P // 2, P // 8
    r1sq, r2sq = float(np.float32(0.2 * 0.2)), float(np.float32(0.4 * 0.4))
    xb = x.reshape(B, P, -1)
    pb = pos.reshape(B, P, 3)

    q1x, q1y, q1z, q2x, q2y, q2z = _run_fps(pb, S1, S2, interpret=interpret)
    q1 = jnp.stack([q1x, q1y, q1z], axis=-1)
    q2 = jnp.stack([q2x, q2y, q2z], axis=-1)

    pen1 = _run_nbr(q1, pb, K, r1sq, interpret=interpret)
    pen2 = _run_nbr(q2, q1, K, r2sq, interpret=interpret)

    f1 = 128
    zpad = jnp.zeros((B, P, f1 - 6), jnp.float32)
    feat1 = jnp.concatenate([xb, pb, zpad], axis=-1)
    (w1, b1), (w2, b2), (w3, b3) = params['sa1']
    w1pad = jnp.concatenate(
        [w1, jnp.zeros((f1 - 6, w1.shape[1]), jnp.float32)], 0)
    w1p = w1pad[3:7]
    q1f = jnp.concatenate([q1, jnp.zeros((B, S1, 1), jnp.float32)], axis=-1)
    (g1,), cnt1 = _sc_compact_gather(pen1.reshape(B * S1, P),
                                     [feat1.reshape(B * P, f1)], S1, P, K)
    x1 = _run_sa_sparse([g1.reshape(B, S1 * K, f1)], q1f,
                        cnt1.reshape(B, S1, 1), [w1pad], w1p,
                        _row(b1), w2, _row(b2), w3, _row(b3), min(S1, 64), K,
                        interpret=interpret)

    (v1, c1), (v2, c2), (v3, c3) = params['sa2']
    pos2 = jnp.concatenate([q1, jnp.zeros((B, S1, 125), jnp.float32)],
                           axis=-1)
    cdim = x1.shape[-1]
    v1x = v1[:cdim]
    v1p = jnp.concatenate([v1[cdim:], jnp.zeros((1, v1.shape[1]),
                                                jnp.float32)], 0)
    v1p128 = jnp.concatenate([v1[cdim:], jnp.zeros((125, v1.shape[1]),
                                                   jnp.float32)], 0)
    q2f = jnp.concatenate([q2, jnp.zeros((B, S2, 1), jnp.float32)], axis=-1)
    (g2a, g2b), cnt2 = _sc_compact_gather(
        pen2.reshape(B * S2, S1),
        [x1.reshape(B * S1, cdim), pos2.reshape(B * S1, 128)], S2, S1, K)
    x2 = _run_sa_sparse([g2a.reshape(B, S2 * K, cdim),
                         g2b.reshape(B, S2 * K, 128)], q2f,
                        cnt2.reshape(B, S2, 1), [v1x, v1p128], v1p,
                        _row(c1), v2, _row(c2), v3, _row(c3), min(S2, 32), K,
                        interpret=interpret)

    f3 = jnp.concatenate([x2, q2], axis=-1).reshape(B * S2, -1)
    return _run_tail(f3, [(w, _row(b)) for (w, b) in params['sa3']],
                     [(w, _row(b)) for (w, b) in params['head']],
                     B, interpret=interpret)


def kernel(x, pos, batch, params):
    B = 32
    P = x.shape[0] // B
    return _forward_impl(x, pos, params, B, P)
